# Initial kernel scaffold; baseline (speedup 1.0000x reference)
import jax
import jax.numpy as jnp
from jax.experimental import pallas as pl


def kernel(x, edge_index, edge_attr, W1, b1, W2, b2, W3, b3, Wr, br):
    raise NotImplementedError("write your pallas kernel here")


# speedup vs baseline: 21.0398x; 21.0398x over previous
import functools

import jax
import jax.numpy as jnp
from jax import lax
from jax.experimental import pallas as pl
from jax.experimental.pallas import tpu as pltpu
from jax.experimental.pallas import tpu_sc as plsc

N = 10000
E = 320000
K = 4
D_IN = 128
H = 16

NTILE = 16
NP = N + 112
RPT = NP // NTILE
CHUNK = 128
NCH = 160
EPT = NCH * CHUNK
GBUF = 8

_mesh = plsc.VectorSubcoreMesh(
    core_axis_name="c", subcore_axis_name="s", num_cores=1)
_sc_params = pltpu.CompilerParams(use_tc_tiling_on_sc=False)


def _zero_fill(buf, n):
    def body(i, _):
        buf[i, :] = jnp.zeros((H,), jnp.float32)
        return 0
    lax.fori_loop(0, n, body, 0)


@functools.partial(
    pl.kernel,
    out_type=jax.ShapeDtypeStruct((NP, H), jnp.float32),
    mesh=_mesh,
    compiler_params=_sc_params,
    scratch_types=[
        pltpu.VMEM((NCH, CHUNK), jnp.int32),
        pltpu.VMEM((CHUNK, H), jnp.float32),
        pltpu.VMEM((RPT, H), jnp.float32),
        pltpu.VMEM_SHARED((NP, H), jnp.float32),
    ],
)
def _deg(col_hbm, deg_out, col_v, ones_v, buf_v, acc_sh):
    tid = lax.axis_index("s")
    base = tid * RPT
    pltpu.sync_copy(col_hbm.at[tid], col_v)

    def ones_body(i, _):
        ones_v[i, :] = jnp.full((H,), 1.0, jnp.float32)
        return 0
    lax.fori_loop(0, CHUNK, ones_body, 0)
    _zero_fill(buf_v, RPT)
    pltpu.sync_copy(buf_v, acc_sh.at[pl.ds(base, RPT)])
    plsc.subcore_barrier()

    def edge_body(j, _):
        pltpu.sync_copy(ones_v, acc_sh.at[col_v.at[j]], add=True)
        return 0
    lax.fori_loop(0, NCH, edge_body, 0)
    plsc.subcore_barrier()
    pltpu.sync_copy(acc_sh.at[pl.ds(base, RPT)], buf_v)
    pltpu.sync_copy(buf_v, deg_out.at[pl.ds(base, RPT)])


@functools.partial(
    pl.kernel,
    out_type=[
        jax.ShapeDtypeStruct((NP, H), jnp.float32),
        jax.ShapeDtypeStruct((NP, H), jnp.float32),
    ],
    mesh=_mesh,
    compiler_params=_sc_params,
    scratch_types=[
        pltpu.VMEM((NCH, CHUNK), jnp.int32),
        pltpu.VMEM((NCH, CHUNK), jnp.int32),
        pltpu.VMEM((GBUF, CHUNK, H), jnp.float32),
        pltpu.VMEM((RPT, H), jnp.float32),
        pltpu.VMEM((RPT, H), jnp.float32),
        pltpu.VMEM((RPT, H), jnp.float32),
        pltpu.VMEM((RPT, H), jnp.float32),
        pltpu.VMEM((RPT, H), jnp.float32),
        pltpu.VMEM_SHARED((NP, H), jnp.float32),
        pltpu.SemaphoreType.DMA,
    ],
)
def _layer(row_hbm, col_hbm, dis_hbm, y_hbm, h_out, u_out,
           row_v, col_v, gbuf_v, acc_v, y_v, dis_v, z_v, zero_v,
           acc_sh, gsem):
    tid = lax.axis_index("s")
    base = tid * RPT
    pltpu.sync_copy(row_hbm.at[tid], row_v)
    pltpu.sync_copy(col_hbm.at[tid], col_v)
    pltpu.sync_copy(dis_hbm.at[pl.ds(base, RPT)], dis_v)
    _zero_fill(zero_v, RPT)

    pltpu.sync_copy(y_hbm.at[K].at[pl.ds(base, RPT)], y_v)

    def u0_body(i, _):
        z_v[i, :] = dis_v[i, :] * y_v[i, :]
        return 0
    lax.fori_loop(0, RPT, u0_body, 0)
    pltpu.sync_copy(z_v, u_out.at[pl.ds(base, RPT)])

    for k in range(K - 1, -1, -1):
        pltpu.sync_copy(zero_v, acc_sh.at[pl.ds(base, RPT)])
        plsc.subcore_barrier()

        def edge_group(g, _):
            descs = []
            for b in range(GBUF):
                descs.append(pltpu.async_copy(
                    u_out.at[row_v.at[g * GBUF + b]], gbuf_v.at[b], gsem))
            for d in descs:
                d.wait()
            for b in range(GBUF):
                pltpu.sync_copy(
                    gbuf_v.at[b], acc_sh.at[col_v.at[g * GBUF + b]], add=True)
            return 0
        lax.fori_loop(0, NCH // GBUF, edge_group, 0)
        plsc.subcore_barrier()

        pltpu.sync_copy(acc_sh.at[pl.ds(base, RPT)], acc_v)
        pltpu.sync_copy(y_hbm.at[k].at[pl.ds(base, RPT)], y_v)
        if k > 0:
            def node_body(i, _):
                zi = dis_v[i, :] * acc_v[i, :] + y_v[i, :]
                z_v[i, :] = dis_v[i, :] * zi
                return 0
            lax.fori_loop(0, RPT, node_body, 0)
            pltpu.sync_copy(z_v, u_out.at[pl.ds(base, RPT)])
        else:
            def node0_body(i, _):
                z_v[i, :] = dis_v[i, :] * acc_v[i, :] + y_v[i, :]
                return 0
            lax.fori_loop(0, RPT, node0_body, 0)
            pltpu.sync_copy(z_v, h_out.at[pl.ds(base, RPT)])


RB = NP // 8


def _prep_body(x_ref, w_ref, degr_ref, dis_ref, y_ref):
    deg = degr_ref[...]
    ridx = (pl.program_id(0) * RB
            + lax.broadcasted_iota(jnp.int32, (RB, H), 0))
    safe = jnp.where(deg > 0, deg, 1.0)
    dis_ref[...] = jnp.where((deg > 0) & (ridx < N), lax.rsqrt(safe), 0.0)
    xv = x_ref[...]
    for k in range(K + 1):
        y_ref[k, :, :] = jnp.dot(
            xv, w_ref[k, :, :], preferred_element_type=jnp.float32,
            precision=lax.Precision.HIGHEST)


_prep = pl.pallas_call(
    _prep_body,
    grid=(NP // RB,),
    in_specs=[
        pl.BlockSpec((RB, D_IN), lambda i: (i, 0)),
        pl.BlockSpec((K + 1, D_IN, H), lambda i: (0, 0, 0)),
        pl.BlockSpec((RB, H), lambda i: (i, 0)),
    ],
    out_specs=[
        pl.BlockSpec((RB, H), lambda i: (i, 0)),
        pl.BlockSpec((K + 1, RB, H), lambda i: (0, i, 0)),
    ],
    out_shape=[
        jax.ShapeDtypeStruct((NP, H), jnp.float32),
        jax.ShapeDtypeStruct((K + 1, NP, H), jnp.float32),
    ],
)


def _inter_body(h_ref, b_ref, w_ref, y_ref):
    hv = h_ref[...] + b_ref[...]
    hv = jnp.where(hv >= 0, hv, 0.01 * hv)
    for k in range(K + 1):
        y_ref[k, :, :] = jnp.dot(
            hv, w_ref[k, :, :], preferred_element_type=jnp.float32,
            precision=lax.Precision.HIGHEST)


_inter = pl.pallas_call(
    _inter_body,
    grid=(NP // RB,),
    in_specs=[
        pl.BlockSpec((RB, H), lambda i: (i, 0)),
        pl.BlockSpec((1, H), lambda i: (0, 0)),
        pl.BlockSpec((K + 1, H, H), lambda i: (0, 0, 0)),
    ],
    out_specs=pl.BlockSpec((K + 1, RB, H), lambda i: (0, i, 0)),
    out_shape=jax.ShapeDtypeStruct((K + 1, NP, H), jnp.float32),
)


def _head_body(h_ref, b_ref, wr_ref, br_ref, o_ref):
    hv = h_ref[...] + b_ref[...]
    o_ref[...] = jnp.dot(
        hv, wr_ref[...], preferred_element_type=jnp.float32,
        precision=lax.Precision.HIGHEST) + br_ref[...]


_head = pl.pallas_call(
    _head_body,
    grid=(NP // RB,),
    in_specs=[
        pl.BlockSpec((RB, H), lambda i: (i, 0)),
        pl.BlockSpec((1, H), lambda i: (0, 0)),
        pl.BlockSpec((H, 1), lambda i: (0, 0)),
        pl.BlockSpec((1, 1), lambda i: (0, 0)),
    ],
    out_specs=pl.BlockSpec((RB, 1), lambda i: (i, 0)),
    out_shape=jax.ShapeDtypeStruct((NP, 1), jnp.float32),
)


def kernel(x, edge_index, edge_attr, W1, b1, W2, b2, W3, b3, Wr, br):
    del edge_attr
    row, col = edge_index[0], edge_index[1]
    ept_true = E // NTILE
    pad = EPT - ept_true
    pad_idx = (N + (jnp.arange(pad, dtype=jnp.int32) % (NP - N)))

    def part(idx):
        r = idx.reshape(NTILE, ept_true)
        p = jnp.broadcast_to(pad_idx[None, :], (NTILE, pad))
        return jnp.concatenate([r, p], axis=1).reshape(NTILE, NCH, CHUNK)

    row_p = part(row)
    col_p = part(col)
    x_pad = jnp.pad(x, ((0, NP - N), (0, 0)))

    deg_rows = _deg(col_p)
    dis_exp, y1 = _prep(x_pad, W1, deg_rows)
    h1, _ = _layer(row_p, col_p, dis_exp, y1)
    y2 = _inter(h1, b1.reshape(1, H), W2)
    h2, _ = _layer(row_p, col_p, dis_exp, y2)
    y3 = _inter(h2, b2.reshape(1, H), W3)
    h3, _ = _layer(row_p, col_p, dis_exp, y3)
    out = _head(h3, b3.reshape(1, H), Wr, br.reshape(1, 1))
    return out[:N]
